# Initial kernel scaffold; baseline (speedup 1.0000x reference)
import jax
import jax.numpy as jnp
from jax.experimental import pallas as pl


def kernel(x, edge_index, Wl1, bl1, Wr1, Wl2, bl2, Wr2, Wl3, bl3, Wr3):
    raise NotImplementedError("write your pallas kernel here")


# speedup vs baseline: 9.4358x; 9.4358x over previous
import functools

import jax
import jax.numpy as jnp
from jax import lax
from jax.experimental import pallas as pl
from jax.experimental.pallas import tpu as pltpu
from jax.experimental.pallas import tpu_sc as plsc

N = 10000
E = 320000
D = 128
NC = 2
NS = 16
NW = NC * NS
EPT = E // NW
K = 80
NCHUNK = EPT // K
SCH = 25
NSUP = NCHUNK // SCH
NP = 10240
RPT = NP // NS
NZ = RPT // K
CW = 16

_mesh = plsc.VectorSubcoreMesh(core_axis_name="c", subcore_axis_name="s")


def _agg_body(W, nbuf, h_hbm, esrc_hbm, edst_hbm, p_hbm, *refs):
    (src_v, dst_v, dstc_v, rowidx_v, stage_v) = refs[:5]
    sidx = refs[5:5 + nbuf]
    sems = refs[5 + nbuf:5 + 2 * nbuf]
    acc_sh = refs[-1]
    cid = lax.axis_index("c")
    sid = lax.axis_index("s")
    wid = cid * NS + sid
    row0 = sid * RPT
    z16 = jnp.zeros((16,), jnp.float32)
    i16 = lax.iota(jnp.int32, 16)

    def set_rowidx(base):
        for g in range(K // 16):
            rowidx_v[pl.ds(g * 16, 16)] = base + g * 16 + i16

    def zrow(i, _):
        for g in range(W // 16):
            stage_v[0, i, pl.ds(g * 16, 16)] = z16
        return 0

    lax.fori_loop(0, K, zrow, 0)

    def zchunk(r, _):
        set_rowidx(row0 + r * K)
        pltpu.sync_copy(stage_v.at[0], acc_sh.at[rowidx_v])
        return 0

    lax.fori_loop(0, NZ, zchunk, 0)
    plsc.subcore_barrier()

    def load_idx(b, j):
        for g in range(K // 16):
            sidx[b][pl.ds(g * 16, 16)] = src_v[pl.ds(j * K + g * 16, 16)]

    def load_dst(j):
        for g in range(K // 16):
            dstc_v[pl.ds(g * 16, 16)] = dst_v[pl.ds(j * K + g * 16, 16)]

    ebase = wid * EPT

    def supblock(s, _):
        soff = ebase + s * SCH * K
        pltpu.sync_copy(esrc_hbm.at[pl.ds(soff, SCH * K)], src_v)
        pltpu.sync_copy(edst_hbm.at[pl.ds(soff, SCH * K)], dst_v)

        if nbuf == 1:
            def chunk(j, _):
                load_idx(0, j)
                pltpu.async_copy(h_hbm.at[sidx[0]], stage_v.at[0],
                                 sems[0]).wait()
                load_dst(j)
                pltpu.sync_copy(stage_v.at[0], acc_sh.at[dstc_v], add=True)
                return 0

            lax.fori_loop(0, SCH, chunk, 0)
        else:
            for b in range(2):
                load_idx(b, b)
                pltpu.async_copy(h_hbm.at[sidx[b]], stage_v.at[b], sems[b])

            def ring(i, _):
                for b in range(2):
                    j = 2 * i + b
                    pltpu.make_async_copy(h_hbm.at[sidx[b]],
                                          stage_v.at[b], sems[b]).wait()
                    load_dst(j)
                    pltpu.sync_copy(stage_v.at[b], acc_sh.at[dstc_v],
                                    add=True)

                    @pl.when(j + 2 < SCH)
                    def _fire():
                        load_idx(b, j + 2)
                        pltpu.async_copy(h_hbm.at[sidx[b]], stage_v.at[b],
                                         sems[b])
                return 0

            lax.fori_loop(0, SCH // 2, ring, 0)
            jt = SCH - 1
            pltpu.make_async_copy(h_hbm.at[sidx[0]], stage_v.at[0],
                                  sems[0]).wait()
            load_dst(jt)
            pltpu.sync_copy(stage_v.at[0], acc_sh.at[dstc_v], add=True)
        return 0

    lax.fori_loop(0, NSUP, supblock, 0)
    plsc.subcore_barrier()

    for r in range(NZ):
        rr = row0 + r * K
        set_rowidx(rr)
        pltpu.async_copy(acc_sh.at[rowidx_v], stage_v.at[0], sems[0]).wait()
        pltpu.sync_copy(stage_v.at[0], p_hbm.at[cid, pl.ds(rr, K)])


def _make_agg(W, nbuf):
    scratch = [
        pltpu.VMEM((SCH * K,), jnp.int32),
        pltpu.VMEM((SCH * K,), jnp.int32),
        pltpu.VMEM((K,), jnp.int32),
        pltpu.VMEM((K,), jnp.int32),
        pltpu.VMEM((nbuf, K, W), jnp.float32),
    ]
    scratch += [pltpu.VMEM((K,), jnp.int32)] * nbuf
    scratch += [pltpu.SemaphoreType.DMA] * nbuf
    scratch.append(pltpu.VMEM_SHARED((NP, W), jnp.float32))
    return pl.kernel(functools.partial(_agg_body, W, nbuf),
                     out_type=jax.ShapeDtypeStruct((NC, NP, W), jnp.float32),
                     mesh=_mesh, scratch_types=tuple(scratch))


_agg = _make_agg(D, 2)


def _cnt_body(edst_hbm, dep_hbm, c_hbm, dst_v, dstc_v, rowidx_v, ones_v,
              cacc_sh):
    cid = lax.axis_index("c")
    sid = lax.axis_index("s")
    wid = cid * NS + sid
    row0 = sid * RPT
    z16 = jnp.zeros((16,), jnp.float32)
    one16 = jnp.ones((16,), jnp.float32)
    i16 = lax.iota(jnp.int32, 16)

    def set_rowidx(base):
        for g in range(K // 16):
            rowidx_v[pl.ds(g * 16, 16)] = base + g * 16 + i16

    def fill(val):
        def row(i, _):
            for g in range(D // 16):
                ones_v[i, pl.ds(g * 16, 16)] = val
            return 0

        lax.fori_loop(0, K, row, 0)

    fill(z16)

    def zchunk(r, _):
        set_rowidx(row0 + r * K)
        pltpu.sync_copy(ones_v, cacc_sh.at[rowidx_v])
        return 0

    lax.fori_loop(0, NZ, zchunk, 0)
    fill(one16)
    plsc.subcore_barrier()

    def load_dst(j):
        for g in range(K // 16):
            dstc_v[pl.ds(g * 16, 16)] = dst_v[pl.ds(j * K + g * 16, 16)]

    ebase = wid * EPT

    def supblock(s, _):
        soff = ebase + s * SCH * K
        pltpu.sync_copy(edst_hbm.at[pl.ds(soff, SCH * K)], dst_v)

        def chunk(j, _):
            load_dst(j)
            pltpu.sync_copy(ones_v, cacc_sh.at[dstc_v], add=True)
            return 0

        lax.fori_loop(0, SCH, chunk, 0)
        return 0

    lax.fori_loop(0, NSUP, supblock, 0)
    plsc.subcore_barrier()

    for r in range(NZ):
        rr = row0 + r * K
        set_rowidx(rr)
        pltpu.sync_copy(cacc_sh.at[rowidx_v], ones_v)
        pltpu.sync_copy(ones_v, c_hbm.at[cid, pl.ds(rr, K)])


_cnt = pl.kernel(
    _cnt_body,
    out_type=jax.ShapeDtypeStruct((NC, NP, D), jnp.float32),
    mesh=_mesh,
    scratch_types=(
        pltpu.VMEM((SCH * K,), jnp.int32),
        pltpu.VMEM((K,), jnp.int32),
        pltpu.VMEM((K,), jnp.int32),
        pltpu.VMEM((K, D), jnp.float32),
        pltpu.VMEM_SHARED((NP, D), jnp.float32),
    ),
)

RB = 2000


def _dense_body(act, W, p_ref, c_ref, h_ref, wlt_ref, bl_ref, wrt_ref,
                o_ref):
    s = p_ref[0, :, :D] + p_ref[1, :, :D]
    cnt = c_ref[0, :, 0:1] + c_ref[1, :, 0:1]
    mean = s * (1.0 / jnp.maximum(cnt, 1.0))
    y = (jnp.dot(mean, wlt_ref[...], preferred_element_type=jnp.float32)
         + bl_ref[...]
         + jnp.dot(h_ref[...], wrt_ref[...], preferred_element_type=jnp.float32))
    o_ref[...] = act(y)


def _dense(act, p, c, h, Wl, bl, Wr):
    W = p.shape[-1]
    return pl.pallas_call(
        functools.partial(_dense_body, act, W),
        grid=(N // RB,),
        in_specs=[
            pl.BlockSpec((NC, RB, W), lambda i: (0, i, 0)),
            pl.BlockSpec((NC, RB, D), lambda i: (0, i, 0)),
            pl.BlockSpec((RB, D), lambda i: (i, 0)),
            pl.BlockSpec((D, D), lambda i: (0, 0)),
            pl.BlockSpec((1, D), lambda i: (0, 0)),
            pl.BlockSpec((D, D), lambda i: (0, 0)),
        ],
        out_specs=pl.BlockSpec((RB, D), lambda i: (i, 0)),
        out_shape=jax.ShapeDtypeStruct((N, D), jnp.float32),
    )(p, c, h, Wl.T, bl.reshape(1, D), Wr.T)


def kernel(x, edge_index, Wl1, bl1, Wr1, Wl2, bl2, Wr2, Wl3, bl3, Wr3):
    esrc = edge_index[0]
    edst = edge_index[1]
    p1 = _agg(x, esrc, edst)
    c = _cnt(edst, p1)
    h1 = _dense(jax.nn.relu, p1, c, x, Wl1, bl1, Wr1)
    p2 = _agg(h1, esrc, edst)
    h2 = _dense(jax.nn.relu, p2, c, h1, Wl2, bl2, Wr2)
    p3 = _agg(h2, esrc, edst)
    h3 = _dense(jax.nn.sigmoid, p3, c, h2, Wl3, bl3, Wr3)
    return h3
